# Initial kernel scaffold; baseline (speedup 1.0000x reference)
import jax
import jax.numpy as jnp
from jax import lax
from jax.experimental import pallas as pl
from jax.experimental.pallas import tpu as pltpu
from jax.experimental.pallas import tpu_sc as plsc

_N = 1024
_D = 256
_L = 16
_C = _N // _L
_ROWS_PER_TILE = _N // 16
_BIG = jnp.int32(2**30)


def _cost_body(x_ref, y_ref, out_ref):
    x = x_ref[...]
    y = y_ref[...]
    xx = jnp.sum(x * x, axis=1)[:, None]
    yy = jnp.sum(y * y, axis=1)[None, :]
    xy = lax.dot_general(x, y, (((1,), (1,)), ((), ())),
                         preferred_element_type=jnp.float32)
    d2 = xx + yy - 2.0 * xy
    out_ref[...] = jnp.maximum(d2, 0.0)


def _cost_matrix(x, y):
    return pl.pallas_call(
        _cost_body,
        out_shape=jax.ShapeDtypeStruct((_N, _N), jnp.float32),
    )(x, y)


def _lanes():
    return lax.iota(jnp.int32, 16)


def _argmin_chunks(read_chunk):
    lanes = _lanes()

    def body(c, carry):
        bv, bi = carry
        v = read_chunk(c)
        idx = lanes + c * _L
        better = v < bv
        return (jnp.where(better, v, bv), jnp.where(better, idx, bi))

    bv, bi = lax.fori_loop(
        0, _C, body,
        (jnp.full((16,), jnp.inf, jnp.float32), jnp.full((16,), _BIG)))
    m = jnp.min(bv, axis=0)
    flat = jnp.min(jnp.where(bv == m, bi, _BIG), axis=0)
    return m, flat


def _scatter1(ref, flat_idx, value):
    lane0 = _lanes() == 0
    plsc.store_scatter(
        ref,
        [jnp.full((16,), flat_idx // _L), jnp.full((16,), flat_idx % _L)],
        jnp.full((16,), value, ref.dtype),
        mask=lane0)


def _gather1(ref, flat_idx):
    v = plsc.load_gather(
        ref,
        [jnp.full((16,), flat_idx // _L), jnp.full((16,), flat_idx % _L)])
    return jnp.min(v, axis=0)


def _match_body(cost_hbm, out_hbm, initbuf, part_v, part_c,
                rmv_sh, rmc_sh, rmv_v, rmc_v, colpen_v, r2c_v, rowbuf):
    cid = lax.axis_index("c")
    sid = lax.axis_index("s")

    @pl.when(cid == 0)
    def _init():
        base = sid * _ROWS_PER_TILE
        pltpu.sync_copy(cost_hbm.at[pl.ds(base, _ROWS_PER_TILE)], initbuf)

        def per_row(r, _):
            m, flat = _argmin_chunks(lambda c: initbuf[r, c])
            _scatter1(part_v, r, m)
            _scatter1(part_c, r, flat)
            return 0

        lax.fori_loop(0, _ROWS_PER_TILE, per_row, 0)
        nch = _ROWS_PER_TILE // _L
        pltpu.sync_copy(part_v, rmv_sh.at[pl.ds(sid * nch, nch)])
        pltpu.sync_copy(part_c, rmc_sh.at[pl.ds(sid * nch, nch)])

    plsc.subcore_barrier()

    @pl.when((cid == 0) & (sid == 0))
    def _greedy():
        pltpu.sync_copy(rmv_sh, rmv_v)
        pltpu.sync_copy(rmc_sh, rmc_v)

        def zero_pen(c, _):
            colpen_v[c] = jnp.zeros((16,), jnp.float32)
            return 0

        lax.fori_loop(0, _C, zero_pen, 0)

        def cond(assigned):
            return assigned < _N

        def step(assigned):
            _, i = _argmin_chunks(lambda c: rmv_v[c])
            j = _gather1(rmc_v, i)
            pen_j = _gather1(colpen_v, j)

            def assign(a):
                _scatter1(r2c_v, i, j)
                _scatter1(rmv_v, i, jnp.float32(jnp.inf))
                _scatter1(colpen_v, j, jnp.float32(jnp.inf))
                return a + 1

            def rescan(a):
                pltpu.sync_copy(cost_hbm.at[i], rowbuf)
                nm, nc = _argmin_chunks(lambda c: rowbuf[c] + colpen_v[c])
                _scatter1(rmv_v, i, nm)
                _scatter1(rmc_v, i, nc)
                return a

            return lax.cond(pen_j == 0.0, assign, rescan, assigned)

        lax.while_loop(cond, step, jnp.int32(0))
        pltpu.sync_copy(r2c_v, out_hbm)


def _greedy_match_sc(cost3):
    mesh = plsc.VectorSubcoreMesh(core_axis_name="c", subcore_axis_name="s")
    nch = _ROWS_PER_TILE // _L
    k = pl.kernel(
        _match_body,
        out_type=jax.ShapeDtypeStruct((_C, _L), jnp.int32),
        mesh=mesh,
        scratch_types=[
            pltpu.VMEM((_ROWS_PER_TILE, _C, _L), jnp.float32),
            pltpu.VMEM((nch, _L), jnp.float32),
            pltpu.VMEM((nch, _L), jnp.int32),
            pltpu.VMEM_SHARED((_C, _L), jnp.float32),
            pltpu.VMEM_SHARED((_C, _L), jnp.int32),
            pltpu.VMEM((_C, _L), jnp.float32),
            pltpu.VMEM((_C, _L), jnp.int32),
            pltpu.VMEM((_C, _L), jnp.float32),
            pltpu.VMEM((_C, _L), jnp.int32),
            pltpu.VMEM((_C, _L), jnp.float32),
        ],
    )
    return k(cost3)


def kernel(xINP, yINP):
    cost = _cost_matrix(xINP, yINP)
    r2c = _greedy_match_sc(cost.reshape(_N, _C, _L))
    return r2c.reshape(_N)


# speedup vs baseline: 294.0609x; 294.0609x over previous
import jax
import jax.numpy as jnp
from jax import lax
from jax.experimental import pallas as pl
from jax.experimental.pallas import tpu as pltpu
from jax.experimental.pallas import tpu_sc as plsc

_N = 1024
_D = 256
_L = 16
_C = _N // _L
_ROWS_PER_TILE = _N // 16
_BIG = 2**30


def _cost_body(x_ref, y_ref, out_ref):
    x = x_ref[...]
    y = y_ref[...]
    xx = jnp.sum(x * x, axis=1)[:, None]
    yy = jnp.sum(y * y, axis=1)[None, :]
    xy = lax.dot_general(x, y, (((1,), (1,)), ((), ())),
                         preferred_element_type=jnp.float32)
    d2 = xx + yy - 2.0 * xy
    out_ref[...] = jnp.maximum(d2, 0.0)


def _cost_matrix(x, y):
    return pl.pallas_call(
        _cost_body,
        out_shape=jax.ShapeDtypeStruct((_N, _N), jnp.float32),
    )(x, y)


def _lanes():
    return lax.iota(jnp.int32, 16)


def _argmin_chunks(read_chunk):
    lanes = _lanes()

    def body(c, carry):
        bv, bi = carry
        v = read_chunk(c)
        idx = lanes + c * _L
        better = v < bv
        return (jnp.where(better, v, bv), jnp.where(better, idx, bi))

    bv, bi = lax.fori_loop(
        0, _C, body,
        (jnp.full((16,), jnp.inf, jnp.float32),
         jnp.full((16,), _BIG, jnp.int32)))
    m = jnp.min(bv, axis=0)
    flat = jnp.min(jnp.where(bv == m, bi, _BIG), axis=0)
    return m, flat


def _scatter1(ref, flat_idx, value):
    lane0 = _lanes() == 0
    plsc.store_scatter(
        ref,
        [jnp.full((16,), flat_idx // _L), jnp.full((16,), flat_idx % _L)],
        jnp.full((16,), value, ref.dtype),
        mask=lane0)


def _gather1(ref, flat_idx):
    v = plsc.load_gather(
        ref,
        [jnp.full((16,), flat_idx // _L), jnp.full((16,), flat_idx % _L)])
    return jnp.min(v, axis=0)


def _match_body(cost_hbm, out_hbm, initbuf, part_v, part_c,
                rmv_sh, rmc_sh, rmv_v, rmc_v, colpen_v, r2c_v, rowbuf):
    cid = lax.axis_index("c")
    sid = lax.axis_index("s")

    @pl.when(cid == 0)
    def _init():
        base = sid * _ROWS_PER_TILE
        pltpu.sync_copy(cost_hbm.at[pl.ds(base, _ROWS_PER_TILE)], initbuf)

        def per_row(r, _):
            m, flat = _argmin_chunks(lambda c: initbuf[r, c])
            _scatter1(part_v, r, m)
            _scatter1(part_c, r, flat)
            return 0

        lax.fori_loop(0, _ROWS_PER_TILE, per_row, 0)
        nch = _ROWS_PER_TILE // _L
        pltpu.sync_copy(part_v, rmv_sh.at[pl.ds(sid * nch, nch)])
        pltpu.sync_copy(part_c, rmc_sh.at[pl.ds(sid * nch, nch)])

    plsc.subcore_barrier()

    @pl.when((cid == 0) & (sid == 0))
    def _greedy():
        pltpu.sync_copy(rmv_sh, rmv_v)
        pltpu.sync_copy(rmc_sh, rmc_v)

        def zero_pen(c, _):
            colpen_v[c] = jnp.zeros((16,), jnp.float32)
            return 0

        lax.fori_loop(0, _C, zero_pen, 0)

        def cond(assigned):
            return assigned < _N

        def step(assigned):
            _, i = _argmin_chunks(lambda c: rmv_v[c])
            j = _gather1(rmc_v, i)
            pen_j = _gather1(colpen_v, j)

            def assign(a):
                _scatter1(r2c_v, i, j)
                _scatter1(rmv_v, i, jnp.float32(jnp.inf))
                _scatter1(colpen_v, j, jnp.float32(jnp.inf))
                return a + 1

            def rescan(a):
                pltpu.sync_copy(cost_hbm.at[i], rowbuf)
                nm, nc = _argmin_chunks(lambda c: rowbuf[c] + colpen_v[c])
                _scatter1(rmv_v, i, nm)
                _scatter1(rmc_v, i, nc)
                return a

            return lax.cond(pen_j == 0.0, assign, rescan, assigned)

        lax.while_loop(cond, step, jnp.int32(0))
        pltpu.sync_copy(r2c_v, out_hbm)


def _greedy_match_sc(cost3):
    mesh = plsc.VectorSubcoreMesh(core_axis_name="c", subcore_axis_name="s",
                                  num_cores=2, num_subcores=16)
    nch = _ROWS_PER_TILE // _L
    k = pl.kernel(
        _match_body,
        out_type=jax.ShapeDtypeStruct((_C, _L), jnp.int32),
        mesh=mesh,
        scratch_types=[
            pltpu.VMEM((_ROWS_PER_TILE, _C, _L), jnp.float32),
            pltpu.VMEM((nch, _L), jnp.float32),
            pltpu.VMEM((nch, _L), jnp.int32),
            pltpu.VMEM_SHARED((_C, _L), jnp.float32),
            pltpu.VMEM_SHARED((_C, _L), jnp.int32),
            pltpu.VMEM((_C, _L), jnp.float32),
            pltpu.VMEM((_C, _L), jnp.int32),
            pltpu.VMEM((_C, _L), jnp.float32),
            pltpu.VMEM((_C, _L), jnp.int32),
            pltpu.VMEM((_C, _L), jnp.float32),
        ],
        compiler_params=pltpu.CompilerParams(needs_layout_passes=False,
                                             use_tc_tiling_on_sc=False),
    )
    return k(cost3)


def kernel(xINP, yINP):
    cost = _cost_matrix(xINP, yINP)
    r2c = _greedy_match_sc(cost.reshape(_N, _C, _L))
    return r2c.reshape(_N)


# speedup vs baseline: 789.5414x; 2.6850x over previous
import jax
import jax.numpy as jnp
from jax import lax
from jax.experimental import pallas as pl
from jax.experimental.pallas import tpu as pltpu
from jax.experimental.pallas import tpu_sc as plsc

_N = 1024
_D = 256
_L = 16
_C = _N // _L
_MC = _C // _L
_ROWS_PER_TILE = _N // 16
_BIG = 2**30
_SH_BUFS = 7
_SH_BLOCK = 128
_SH_ROWS = _SH_BUFS * _SH_BLOCK


def _cost_body(x_ref, y_ref, out_ref):
    x = x_ref[...]
    y = y_ref[...]
    xx = jnp.sum(x * x, axis=1)[:, None]
    yy = jnp.sum(y * y, axis=1)[None, :]
    xy = lax.dot_general(x, y, (((1,), (1,)), ((), ())),
                         preferred_element_type=jnp.float32)
    d2 = xx + yy - 2.0 * xy
    out_ref[...] = jnp.maximum(d2, 0.0)


def _cost_matrix(x, y):
    return pl.pallas_call(
        _cost_body,
        out_shape=jax.ShapeDtypeStruct((_N, _N), jnp.float32),
    )(x, y)


def _lanes():
    return lax.iota(jnp.int32, 16)


def _argmin_chunks(read_chunk, n_chunks=_C, unroll=8):
    lanes = _lanes()

    def body(c, carry):
        bv, bi = carry
        v = read_chunk(c)
        idx = lanes + c * _L
        better = v < bv
        return (jnp.where(better, v, bv), jnp.where(better, idx, bi))

    bv, bi = lax.fori_loop(
        0, n_chunks, body,
        (jnp.full((16,), jnp.inf, jnp.float32),
         jnp.full((16,), _BIG, jnp.int32)),
        unroll=unroll)
    m = jnp.min(bv, axis=0)
    flat = jnp.min(jnp.where(bv == m, bi, _BIG), axis=0)
    return m, flat


def _scatter1(ref, flat_idx, value):
    lane0 = _lanes() == 0
    plsc.store_scatter(
        ref,
        [jnp.full((16,), flat_idx // _L), jnp.full((16,), flat_idx % _L)],
        jnp.full((16,), value, ref.dtype),
        mask=lane0)


def _gather1(ref, flat_idx):
    v = plsc.load_gather(
        ref,
        [jnp.full((16,), flat_idx // _L), jnp.full((16,), flat_idx % _L)])
    return jnp.min(v, axis=0)


def _match_body(cost_hbm, out_hbm, initbuf, part_v, part_c,
                rmv_sh, rmc_sh, *rest):
    cost_sh = rest[:_SH_BUFS]
    (rmv_v, rmc_v, cmin_v, colpen_v, r2c_v, rowbuf) = rest[_SH_BUFS:]
    cid = lax.axis_index("c")
    sid = lax.axis_index("s")

    @pl.when(cid == 0)
    def _init():
        base = sid * _ROWS_PER_TILE
        pltpu.sync_copy(cost_hbm.at[pl.ds(base, _ROWS_PER_TILE)], initbuf)
        for bb in range(_SH_BUFS):
            @pl.when(sid // 2 == bb)
            def _stage(bb=bb):
                half = (sid % 2) * _ROWS_PER_TILE
                pltpu.sync_copy(initbuf,
                                cost_sh[bb].at[pl.ds(half, _ROWS_PER_TILE)])

        def per_row(r, _):
            m, flat = _argmin_chunks(lambda c: initbuf[r, c])
            _scatter1(part_v, r, m)
            _scatter1(part_c, r, flat)
            return 0

        lax.fori_loop(0, _ROWS_PER_TILE, per_row, 0)
        nch = _ROWS_PER_TILE // _L
        pltpu.sync_copy(part_v, rmv_sh.at[pl.ds(sid * nch, nch)])
        pltpu.sync_copy(part_c, rmc_sh.at[pl.ds(sid * nch, nch)])

    plsc.subcore_barrier()

    @pl.when((cid == 0) & (sid == 0))
    def _greedy():
        pltpu.sync_copy(rmv_sh, rmv_v)
        pltpu.sync_copy(rmc_sh, rmc_v)

        def init_aux(c, _):
            colpen_v[c] = jnp.zeros((16,), jnp.float32)
            _scatter1(cmin_v, c, jnp.min(rmv_v[c], axis=0))
            return 0

        lax.fori_loop(0, _C, init_aux, 0, unroll=8)

        def refresh_cmin(k):
            _scatter1(cmin_v, k, jnp.min(rmv_v[k], axis=0))

        def cond(assigned):
            return assigned < _N

        def step(assigned):
            m, k = _argmin_chunks(lambda mc: cmin_v[mc],
                                  n_chunks=_MC, unroll=_MC)
            v = rmv_v[k]
            i = jnp.min(jnp.where(v == m, _lanes() + k * _L, _BIG), axis=0)
            j = _gather1(rmc_v, i)
            pen_j = _gather1(colpen_v, j)

            def assign(a):
                _scatter1(r2c_v, i, j)
                _scatter1(rmv_v, i, jnp.float32(jnp.inf))
                _scatter1(colpen_v, j, jnp.float32(jnp.inf))
                refresh_cmin(i // _L)
                return a + 1

            def rescan(a):
                blk = i // _SH_BLOCK
                for bb in range(_SH_BUFS):
                    @pl.when(blk == bb)
                    def _fast(bb=bb):
                        pltpu.sync_copy(cost_sh[bb].at[i % _SH_BLOCK], rowbuf)

                @pl.when(i >= _SH_ROWS)
                def _slow():
                    pltpu.sync_copy(cost_hbm.at[i], rowbuf)

                nm, nc = _argmin_chunks(lambda c: rowbuf[c] + colpen_v[c])
                _scatter1(rmv_v, i, nm)
                _scatter1(rmc_v, i, nc)
                refresh_cmin(i // _L)
                return a

            return lax.cond(pen_j == 0.0, assign, rescan, assigned)

        lax.while_loop(cond, step, jnp.int32(0))
        pltpu.sync_copy(r2c_v, out_hbm)


def _greedy_match_sc(cost3):
    mesh = plsc.VectorSubcoreMesh(core_axis_name="c", subcore_axis_name="s",
                                  num_cores=1, num_subcores=16)
    nch = _ROWS_PER_TILE // _L
    k = pl.kernel(
        _match_body,
        out_type=jax.ShapeDtypeStruct((_C, _L), jnp.int32),
        mesh=mesh,
        scratch_types=[
            pltpu.VMEM((_ROWS_PER_TILE, _C, _L), jnp.float32),
            pltpu.VMEM((nch, _L), jnp.float32),
            pltpu.VMEM((nch, _L), jnp.int32),
            pltpu.VMEM_SHARED((_C, _L), jnp.float32),
            pltpu.VMEM_SHARED((_C, _L), jnp.int32),
            *[pltpu.VMEM_SHARED((_SH_BLOCK, _C, _L), jnp.float32)
              for _ in range(_SH_BUFS)],
            pltpu.VMEM((_C, _L), jnp.float32),
            pltpu.VMEM((_C, _L), jnp.int32),
            pltpu.VMEM((_MC, _L), jnp.float32),
            pltpu.VMEM((_C, _L), jnp.float32),
            pltpu.VMEM((_C, _L), jnp.int32),
            pltpu.VMEM((_C, _L), jnp.float32),
        ],
        compiler_params=pltpu.CompilerParams(needs_layout_passes=False,
                                             use_tc_tiling_on_sc=False),
    )
    return k(cost3)


def kernel(xINP, yINP):
    cost = _cost_matrix(xINP, yINP)
    r2c = _greedy_match_sc(cost.reshape(_N, _C, _L))
    return r2c.reshape(_N)
